# baseline (speedup 1.0000x reference)
import functools

import jax
import jax.numpy as jnp
from jax import lax
from jax.experimental import pallas as pl
from jax.experimental.pallas import tpu as pltpu
from jax.experimental.pallas import tpu_sc as plsc

NC = 2
NS = 16
NW = NC * NS
B = 112
DF = 128
NBUF = 3
NB0 = 108
NB1 = 72


def _spmm_sc(pk, x, n_pad):
    rows_per_tile = n_pad // NS
    mesh = plsc.VectorSubcoreMesh(core_axis_name="c", subcore_axis_name="s")

    @functools.partial(
        pl.kernel,
        out_type=jax.ShapeDtypeStruct((NC, n_pad, DF), jnp.float32),
        mesh=mesh,
        scratch_types=[
            pltpu.VMEM((NBUF, 3, B), jnp.int32),
            pltpu.VMEM((NBUF, B, DF), jnp.float32),
            pltpu.VMEM_SHARED((n_pad, DF), jnp.float32),
            pltpu.SemaphoreType.DMA,
            pltpu.SemaphoreType.DMA,
            pltpu.SemaphoreType.DMA,
        ],
    )
    def k(pk_hbm, x_hbm, out_hbm,
          pk_v, rows_v, acc_sh, isem, gsem, ssem):
        c = lax.axis_index("c")
        s = lax.axis_index("s")
        wid = c * NS + s
        nbc = jnp.where(c == 0, NB0, NB1)

        def zrow(i, carry0):
            for k8 in range(DF // 16):
                rows_v[0, i, pl.ds(k8 * 16, 16)] = jnp.zeros(
                    (16,), jnp.float32)
            return carry0

        lax.fori_loop(0, B, zrow, 0)
        r0 = s * rows_per_tile
        for t in range(rows_per_tile // B):
            pltpu.sync_copy(rows_v.at[0], acc_sh.at[pl.ds(r0 + t * B, B)])
        rem = rows_per_tile % B
        if rem:
            pltpu.sync_copy(
                rows_v.at[0, pl.ds(0, rem)],
                acc_sh.at[pl.ds(r0 + (rows_per_tile // B) * B, rem)])

        def idx_start(j, b):
            pltpu.async_copy(pk_hbm.at[wid, j], pk_v.at[b], isem)

        def idx_wait(b):
            pltpu.make_async_copy(pk_hbm.at[wid, 0], pk_v.at[b], isem).wait()

        def gather_start(b):
            pltpu.async_copy(x_hbm.at[pk_v.at[b, 0]], rows_v.at[b], gsem)

        def gather_wait(b):
            pltpu.make_async_copy(x_hbm.at[pk_v.at[b, 0]], rows_v.at[b],
                                  gsem).wait()

        def scat_start(b):
            pltpu.async_copy(rows_v.at[b], acc_sh.at[pk_v.at[b, 1]], ssem,
                             add=True)

        def scat_wait(b):
            pltpu.make_async_copy(rows_v.at[b], acc_sh.at[pk_v.at[b, 1]],
                                  ssem).wait()

        def scale(b):
            def group_body(g, carry2):
                v16 = lax.bitcast_convert_type(pk_v[b, 2, pl.ds(g * 16, 16)],
                                               jnp.float32)
                for lane in range(16):
                    ev = v16[lane]
                    row = g * 16 + lane
                    for k8 in range(DF // 16):
                        sl = pl.ds(k8 * 16, 16)
                        rows_v[b, row, sl] = rows_v[b, row, sl] * ev
                return carry2

            lax.fori_loop(0, B // 16, group_body, 0)

        idx_start(0, 0)
        plsc.subcore_barrier()
        idx_wait(0)
        gather_start(0)
        idx_start(1, 1)

        def trio_body(jj, carry):
            j0 = jj * NBUF
            for b in range(NBUF):
                j = j0 + b
                gather_wait(b)

                @pl.when(j + 1 < nbc)
                def _():
                    idx_wait((b + 1) % NBUF)
                    gather_start((b + 1) % NBUF)

                scale(b)
                scat_start(b)

                @pl.when(j >= 1)
                def _():
                    scat_wait((b + NBUF - 1) % NBUF)

                @pl.when(j + 2 < nbc)
                def _():
                    idx_start(j + 2, (b + 2) % NBUF)
            return carry

        lax.fori_loop(0, nbc // NBUF, trio_body, 0)
        scat_wait(NBUF - 1)

        plsc.subcore_barrier()
        pltpu.sync_copy(acc_sh.at[pl.ds(r0, rows_per_tile)],
                        out_hbm.at[c, pl.ds(r0, rows_per_tile)])

    return k(pk, x)


def _linear_tc(p, w, b2d, relu, n):
    blk = 1000
    grid = n // blk

    def body(p_ref, w_ref, b_ref, o_ref):
        x = p_ref[0] + p_ref[1]
        y = lax.dot_general(x, w_ref[...],
                            dimension_numbers=(((1,), (1,)), ((), ())),
                            preferred_element_type=jnp.float32)
        y = y + b_ref[...]
        if relu:
            y = jnp.maximum(y, 0.0)
        o_ref[...] = y

    return pl.pallas_call(
        body,
        out_shape=jax.ShapeDtypeStruct((n, w.shape[0]), jnp.float32),
        grid=(grid,),
        in_specs=[
            pl.BlockSpec((2, blk, DF), lambda i: (0, i, 0)),
            pl.BlockSpec((w.shape[0], w.shape[1]), lambda i: (0, 0)),
            pl.BlockSpec((1, w.shape[0]), lambda i: (0, 0)),
        ],
        out_specs=pl.BlockSpec((blk, w.shape[0]), lambda i: (i, 0)),
    )(p, w, b2d)


def kernel(A_indices, A_values, X, W1, b1, W2, b2):
    n = X.shape[0]
    e = A_values.shape[0]
    dst = A_indices[0]
    src = A_indices[1]

    L0, L1 = NB0 * B, NB1 * B
    e_pad = NS * (L0 + L1)
    pad = e_pad - e
    if pad:
        src = jnp.concatenate([src, jnp.zeros((pad,), jnp.int32)])
        dst = jnp.concatenate([dst, jnp.zeros((pad,), jnp.int32)])
        vals = jnp.concatenate([A_values, jnp.zeros((pad,), jnp.float32)])
    else:
        vals = A_values
    nbm = max(NB0, NB1)
    e0 = NS * L0

    def tiled(a):
        a0 = a[:e0].reshape(NS, NB0, B)
        a1 = a[e0:].reshape(NS, NB1, B)
        a0 = jnp.pad(a0, ((0, 0), (0, nbm - NB0), (0, 0)))
        a1 = jnp.pad(a1, ((0, 0), (0, nbm - NB1), (0, 0)))
        return jnp.concatenate([a0, a1], axis=0)

    vbits = lax.bitcast_convert_type(vals, jnp.int32)
    pk = jnp.stack([tiled(src), tiled(dst), tiled(vbits)], axis=2)

    align = 8 * NS
    n_pad = ((n + align - 1) // align) * align
    b1_2d = b1.reshape(1, -1)
    b2_2d = b2.reshape(1, -1)

    p1 = _spmm_sc(pk, X, n_pad)
    h = _linear_tc(p1, W1, b1_2d, relu=True, n=n)
    p2 = _spmm_sc(pk, h, n_pad)
    out = _linear_tc(p2, W2, b2_2d, relu=False, n=n)
    return out


# speedup vs baseline: 1.0641x; 1.0641x over previous
import functools

import jax
import jax.numpy as jnp
from jax import lax
from jax.experimental import pallas as pl
from jax.experimental.pallas import tpu as pltpu
from jax.experimental.pallas import tpu_sc as plsc

NC = 2
NS = 16
NW = NC * NS
B = 112
DF = 128
NBUF = 3
NB0 = 120
NB1 = 60


def _spmm_sc(pk, x, n_pad):
    rows_per_tile = n_pad // NS
    mesh = plsc.VectorSubcoreMesh(core_axis_name="c", subcore_axis_name="s")

    @functools.partial(
        pl.kernel,
        out_type=jax.ShapeDtypeStruct((NC, n_pad, DF), jnp.float32),
        mesh=mesh,
        scratch_types=[
            pltpu.VMEM((NBUF, 3, B), jnp.int32),
            pltpu.VMEM((NBUF, B, DF), jnp.float32),
            pltpu.VMEM_SHARED((n_pad, DF), jnp.float32),
            pltpu.SemaphoreType.DMA,
            pltpu.SemaphoreType.DMA,
            pltpu.SemaphoreType.DMA,
        ],
    )
    def k(pk_hbm, x_hbm, out_hbm,
          pk_v, rows_v, acc_sh, isem, gsem, ssem):
        c = lax.axis_index("c")
        s = lax.axis_index("s")
        wid = c * NS + s
        nbc = jnp.where(c == 0, NB0, NB1)

        def zrow(i, carry0):
            for k8 in range(DF // 16):
                rows_v[0, i, pl.ds(k8 * 16, 16)] = jnp.zeros(
                    (16,), jnp.float32)
            return carry0

        lax.fori_loop(0, B, zrow, 0)
        r0 = s * rows_per_tile
        for t in range(rows_per_tile // B):
            pltpu.sync_copy(rows_v.at[0], acc_sh.at[pl.ds(r0 + t * B, B)])
        rem = rows_per_tile % B
        if rem:
            pltpu.sync_copy(
                rows_v.at[0, pl.ds(0, rem)],
                acc_sh.at[pl.ds(r0 + (rows_per_tile // B) * B, rem)])

        def idx_start(j, b):
            pltpu.async_copy(pk_hbm.at[wid, j], pk_v.at[b], isem)

        def idx_wait(b):
            pltpu.make_async_copy(pk_hbm.at[wid, 0], pk_v.at[b], isem).wait()

        def gather_start(b):
            pltpu.async_copy(x_hbm.at[pk_v.at[b, 0]], rows_v.at[b], gsem)

        def gather_wait(b):
            pltpu.make_async_copy(x_hbm.at[pk_v.at[b, 0]], rows_v.at[b],
                                  gsem).wait()

        def scat_start(b):
            pltpu.async_copy(rows_v.at[b], acc_sh.at[pk_v.at[b, 1]], ssem,
                             add=True)

        def scat_wait(b):
            pltpu.make_async_copy(rows_v.at[b], acc_sh.at[pk_v.at[b, 1]],
                                  ssem).wait()

        def scale(b):
            def group_body(g, carry2):
                v16 = lax.bitcast_convert_type(pk_v[b, 2, pl.ds(g * 16, 16)],
                                               jnp.float32)
                for lane in range(16):
                    ev = v16[lane]
                    row = g * 16 + lane
                    for k8 in range(DF // 16):
                        sl = pl.ds(k8 * 16, 16)
                        rows_v[b, row, sl] = rows_v[b, row, sl] * ev
                return carry2

            lax.fori_loop(0, B // 16, group_body, 0)

        idx_start(0, 0)
        plsc.subcore_barrier()
        idx_wait(0)
        gather_start(0)
        idx_start(1, 1)

        def trio_body(jj, carry):
            j0 = jj * NBUF
            for b in range(NBUF):
                j = j0 + b
                gather_wait(b)

                @pl.when(j + 1 < nbc)
                def _():
                    idx_wait((b + 1) % NBUF)
                    gather_start((b + 1) % NBUF)

                scale(b)
                scat_start(b)

                @pl.when(j >= 1)
                def _():
                    scat_wait((b + NBUF - 1) % NBUF)

                @pl.when(j + 2 < nbc)
                def _():
                    idx_start(j + 2, (b + 2) % NBUF)
            return carry

        lax.fori_loop(0, nbc // NBUF, trio_body, 0)
        scat_wait(NBUF - 1)

        plsc.subcore_barrier()
        pltpu.sync_copy(acc_sh.at[pl.ds(r0, rows_per_tile)],
                        out_hbm.at[c, pl.ds(r0, rows_per_tile)])

    return k(pk, x)


def _linear_tc(p, w, b2d, relu, n):
    blk = 1000
    grid = n // blk

    def body(p_ref, w_ref, b_ref, o_ref):
        x = p_ref[0] + p_ref[1]
        y = lax.dot_general(x, w_ref[...],
                            dimension_numbers=(((1,), (1,)), ((), ())),
                            preferred_element_type=jnp.float32)
        y = y + b_ref[...]
        if relu:
            y = jnp.maximum(y, 0.0)
        o_ref[...] = y

    return pl.pallas_call(
        body,
        out_shape=jax.ShapeDtypeStruct((n, w.shape[0]), jnp.float32),
        grid=(grid,),
        in_specs=[
            pl.BlockSpec((2, blk, DF), lambda i: (0, i, 0)),
            pl.BlockSpec((w.shape[0], w.shape[1]), lambda i: (0, 0)),
            pl.BlockSpec((1, w.shape[0]), lambda i: (0, 0)),
        ],
        out_specs=pl.BlockSpec((blk, w.shape[0]), lambda i: (i, 0)),
    )(p, w, b2d)


def kernel(A_indices, A_values, X, W1, b1, W2, b2):
    n = X.shape[0]
    e = A_values.shape[0]
    dst = A_indices[0]
    src = A_indices[1]

    L0, L1 = NB0 * B, NB1 * B
    e_pad = NS * (L0 + L1)
    pad = e_pad - e
    if pad:
        src = jnp.concatenate([src, jnp.zeros((pad,), jnp.int32)])
        dst = jnp.concatenate([dst, jnp.zeros((pad,), jnp.int32)])
        vals = jnp.concatenate([A_values, jnp.zeros((pad,), jnp.float32)])
    else:
        vals = A_values
    nbm = max(NB0, NB1)
    e0 = NS * L0

    def tiled(a):
        a0 = a[:e0].reshape(NS, NB0, B)
        a1 = a[e0:].reshape(NS, NB1, B)
        a0 = jnp.pad(a0, ((0, 0), (0, nbm - NB0), (0, 0)))
        a1 = jnp.pad(a1, ((0, 0), (0, nbm - NB1), (0, 0)))
        return jnp.concatenate([a0, a1], axis=0)

    vbits = lax.bitcast_convert_type(vals, jnp.int32)
    pk = jnp.stack([tiled(src), tiled(dst), tiled(vbits)], axis=2)

    align = 8 * NS
    n_pad = ((n + align - 1) // align) * align
    b1_2d = b1.reshape(1, -1)
    b2_2d = b2.reshape(1, -1)

    p1 = _spmm_sc(pk, X, n_pad)
    h = _linear_tc(p1, W1, b1_2d, relu=True, n=n)
    p2 = _spmm_sc(pk, h, n_pad)
    out = _linear_tc(p2, W2, b2_2d, relu=False, n=n)
    return out


# speedup vs baseline: 1.0770x; 1.0121x over previous
import functools

import jax
import jax.numpy as jnp
from jax import lax
from jax.experimental import pallas as pl
from jax.experimental.pallas import tpu as pltpu
from jax.experimental.pallas import tpu_sc as plsc

NC = 2
NS = 16
NW = NC * NS
B = 112
DF = 128
NBUF = 3
NB0 = 120
NB1 = 60


def _spmm_sc(pk, x, n_pad):
    rows_per_tile = n_pad // NS
    mesh = plsc.VectorSubcoreMesh(core_axis_name="c", subcore_axis_name="s")

    @functools.partial(
        pl.kernel,
        out_type=jax.ShapeDtypeStruct((NC, n_pad, DF), jnp.float32),
        mesh=mesh,
        scratch_types=[
            pltpu.VMEM((NBUF, 3, B), jnp.int32),
            pltpu.VMEM((NBUF, B, DF), jnp.float32),
            pltpu.VMEM_SHARED((n_pad, DF), jnp.float32),
            pltpu.SemaphoreType.DMA,
            pltpu.SemaphoreType.DMA,
            pltpu.SemaphoreType.DMA,
        ],
    )
    def k(pk_hbm, x_hbm, out_hbm,
          pk_v, rows_v, acc_sh, isem, gsem, ssem):
        c = lax.axis_index("c")
        s = lax.axis_index("s")
        wid = c * NS + s
        nbc = jnp.where(c == 0, NB0, NB1)

        def zrow(i, carry0):
            for k8 in range(DF // 16):
                rows_v[0, i, pl.ds(k8 * 16, 16)] = jnp.zeros(
                    (16,), jnp.float32)
            return carry0

        lax.fori_loop(0, B, zrow, 0)
        r0 = s * rows_per_tile
        for t in range(rows_per_tile // B):
            pltpu.sync_copy(rows_v.at[0], acc_sh.at[pl.ds(r0 + t * B, B)])
        rem = rows_per_tile % B
        if rem:
            pltpu.sync_copy(
                rows_v.at[0, pl.ds(0, rem)],
                acc_sh.at[pl.ds(r0 + (rows_per_tile // B) * B, rem)])

        def idx_start(j, b):
            pltpu.async_copy(pk_hbm.at[wid, j], pk_v.at[b], isem)

        def idx_wait(b):
            pltpu.make_async_copy(pk_hbm.at[wid, 0], pk_v.at[b], isem).wait()

        def gather_start(b):
            pltpu.async_copy(x_hbm.at[pk_v.at[b, 0]], rows_v.at[b], gsem)

        def gather_wait(b):
            pltpu.make_async_copy(x_hbm.at[pk_v.at[b, 0]], rows_v.at[b],
                                  gsem).wait()

        def scat_start(b):
            pltpu.async_copy(rows_v.at[b], acc_sh.at[pk_v.at[b, 1]], ssem,
                             add=True)

        def scat_wait(b):
            pltpu.make_async_copy(rows_v.at[b], acc_sh.at[pk_v.at[b, 1]],
                                  ssem).wait()

        def scale(b):
            def group_body(g, carry2):
                v16 = lax.bitcast_convert_type(pk_v[b, 2, pl.ds(g * 16, 16)],
                                               jnp.float32)
                for lane in range(16):
                    evv = lax.broadcast_in_dim(v16[lane], (16,), ())
                    row = g * 16 + lane
                    for k8 in range(DF // 16):
                        sl = pl.ds(k8 * 16, 16)
                        rows_v[b, row, sl] = rows_v[b, row, sl] * evv
                return carry2

            lax.fori_loop(0, B // 16, group_body, 0)

        idx_start(0, 0)
        plsc.subcore_barrier()
        idx_wait(0)
        gather_start(0)
        idx_start(1, 1)

        def trio_body(jj, carry):
            j0 = jj * NBUF
            for b in range(NBUF):
                j = j0 + b
                gather_wait(b)

                @pl.when(j + 1 < nbc)
                def _():
                    idx_wait((b + 1) % NBUF)
                    gather_start((b + 1) % NBUF)

                scale(b)
                scat_start(b)

                @pl.when(j >= 1)
                def _():
                    scat_wait((b + NBUF - 1) % NBUF)

                @pl.when(j + 2 < nbc)
                def _():
                    idx_start(j + 2, (b + 2) % NBUF)
            return carry

        lax.fori_loop(0, nbc // NBUF, trio_body, 0)
        scat_wait(NBUF - 1)

        plsc.subcore_barrier()
        pltpu.sync_copy(acc_sh.at[pl.ds(r0, rows_per_tile)],
                        out_hbm.at[c, pl.ds(r0, rows_per_tile)])

    return k(pk, x)


def _linear_tc(p, w, b2d, relu, n):
    blk = 1000
    grid = n // blk

    def body(p_ref, w_ref, b_ref, o_ref):
        x = p_ref[0] + p_ref[1]
        y = lax.dot_general(x, w_ref[...],
                            dimension_numbers=(((1,), (1,)), ((), ())),
                            preferred_element_type=jnp.float32)
        y = y + b_ref[...]
        if relu:
            y = jnp.maximum(y, 0.0)
        o_ref[...] = y

    return pl.pallas_call(
        body,
        out_shape=jax.ShapeDtypeStruct((n, w.shape[0]), jnp.float32),
        grid=(grid,),
        in_specs=[
            pl.BlockSpec((2, blk, DF), lambda i: (0, i, 0)),
            pl.BlockSpec((w.shape[0], w.shape[1]), lambda i: (0, 0)),
            pl.BlockSpec((1, w.shape[0]), lambda i: (0, 0)),
        ],
        out_specs=pl.BlockSpec((blk, w.shape[0]), lambda i: (i, 0)),
    )(p, w, b2d)


def kernel(A_indices, A_values, X, W1, b1, W2, b2):
    n = X.shape[0]
    e = A_values.shape[0]
    dst = A_indices[0]
    src = A_indices[1]

    L0, L1 = NB0 * B, NB1 * B
    e_pad = NS * (L0 + L1)
    pad = e_pad - e
    if pad:
        src = jnp.concatenate([src, jnp.zeros((pad,), jnp.int32)])
        dst = jnp.concatenate([dst, jnp.zeros((pad,), jnp.int32)])
        vals = jnp.concatenate([A_values, jnp.zeros((pad,), jnp.float32)])
    else:
        vals = A_values
    nbm = max(NB0, NB1)
    e0 = NS * L0

    def tiled(a):
        a0 = a[:e0].reshape(NS, NB0, B)
        a1 = a[e0:].reshape(NS, NB1, B)
        a0 = jnp.pad(a0, ((0, 0), (0, nbm - NB0), (0, 0)))
        a1 = jnp.pad(a1, ((0, 0), (0, nbm - NB1), (0, 0)))
        return jnp.concatenate([a0, a1], axis=0)

    vbits = lax.bitcast_convert_type(vals, jnp.int32)
    pk = jnp.stack([tiled(src), tiled(dst), tiled(vbits)], axis=2)

    align = 8 * NS
    n_pad = ((n + align - 1) // align) * align
    b1_2d = b1.reshape(1, -1)
    b2_2d = b2.reshape(1, -1)

    p1 = _spmm_sc(pk, X, n_pad)
    h = _linear_tc(p1, W1, b1_2d, relu=True, n=n)
    p2 = _spmm_sc(pk, h, n_pad)
    out = _linear_tc(p2, W2, b2_2d, relu=False, n=n)
    return out
